# Initial kernel scaffold; baseline (speedup 1.0000x reference)
import jax
import jax.numpy as jnp
from jax.experimental import pallas as pl


def kernel(x, edge_index, batch, W1, b1, W2, b2, W3, b3, Wfc, bfc):
    raise NotImplementedError("write your pallas kernel here")


# speedup vs baseline: 15.7426x; 15.7426x over previous
import functools

import jax
import jax.numpy as jnp
from jax import lax
from jax.experimental import pallas as pl
from jax.experimental.pallas import tpu as pltpu
from jax.experimental.pallas import tpu_sc as plsc

N = 50000
E = 800000
D = 64
H = 32
NDUM = 48
NROW = N + NDUM
NTILE = 16
NCORE = 2
RPT = NROW // NTILE
K = 128
NCHUNK = 6400
EPAD = NCHUNK * K
PAD = EPAD - E
CH = 8
DEG_W = 8

_mesh = plsc.VectorSubcoreMesh(
    core_axis_name="c", subcore_axis_name="s",
    num_cores=NCORE, num_subcores=NTILE)

_sc_params = pltpu.CompilerParams(use_tc_tiling_on_sc=False)

f32 = jnp.float32


@functools.partial(
    pl.kernel,
    out_type=(jax.ShapeDtypeStruct((NROW, DEG_W), f32),
              jax.ShapeDtypeStruct((NROW, DEG_W), f32)),
    mesh=_mesh,
    scratch_types=[
        pltpu.VMEM((CH, K), jnp.int32),
        pltpu.VMEM((K, DEG_W), f32),
        pltpu.VMEM_SHARED((NROW, DEG_W), f32),
    ],
    compiler_params=_sc_params,
)
def _deg_kernel(dst2d, ones_hbm, zeros_hbm, deg0, deg1, dstv, onesv, degsh):
    c = lax.axis_index("c")
    s = lax.axis_index("s")
    base = s * RPT
    pltpu.sync_copy(zeros_hbm.at[pl.ds(base, RPT)], degsh.at[pl.ds(base, RPT)])
    pltpu.sync_copy(ones_hbm, onesv)
    plsc.subcore_barrier()

    wid = c * NTILE + s
    chunks_per_w = NCHUNK // (NCORE * NTILE)
    first = wid * chunks_per_w

    def outer(i, carry):
        blk = first + i * CH
        pltpu.sync_copy(dst2d.at[pl.ds(blk, CH)], dstv)
        for j in range(CH):
            pltpu.sync_copy(onesv, degsh.at[dstv.at[j]], add=True)
        return carry

    lax.fori_loop(0, chunks_per_w // CH, outer, 0)
    plsc.subcore_barrier()

    @pl.when(c == 0)
    def _():
        pltpu.sync_copy(degsh.at[pl.ds(base, RPT)], deg0.at[pl.ds(base, RPT)])

    @pl.when(c == 1)
    def _():
        pltpu.sync_copy(degsh.at[pl.ds(base, RPT)], deg1.at[pl.ds(base, RPT)])


@functools.partial(
    pl.kernel,
    out_type=(jax.ShapeDtypeStruct((NROW, H), f32),
              jax.ShapeDtypeStruct((NROW, H), f32)),
    mesh=_mesh,
    scratch_types=[
        pltpu.VMEM((CH, K), jnp.int32),
        pltpu.VMEM((CH, K), jnp.int32),
        pltpu.VMEM((K, H), f32),
        pltpu.VMEM_SHARED((NROW, H), f32),
    ],
    compiler_params=_sc_params,
)
def _agg_kernel(hs0, hs1, src2d, dst2d, agg0, agg1, srcv, dstv, rows, aggsh):
    c = lax.axis_index("c")
    s = lax.axis_index("s")
    base = s * RPT
    chunks_per_t = NCHUNK // NTILE
    first = s * chunks_per_t

    def run(hs, out):
        pltpu.sync_copy(hs.at[pl.ds(base, RPT)], aggsh.at[pl.ds(base, RPT)])
        plsc.subcore_barrier()

        def outer(i, carry):
            blk = first + i * CH
            pltpu.sync_copy(src2d.at[pl.ds(blk, CH)], srcv)
            pltpu.sync_copy(dst2d.at[pl.ds(blk, CH)], dstv)
            for j in range(CH):
                pltpu.sync_copy(hs.at[srcv.at[j]], rows)
                pltpu.sync_copy(rows, aggsh.at[dstv.at[j]], add=True)
            return carry

        lax.fori_loop(0, chunks_per_t // CH, outer, 0)
        plsc.subcore_barrier()
        pltpu.sync_copy(aggsh.at[pl.ds(base, RPT)], out.at[pl.ds(base, RPT)])

    @pl.when(c == 0)
    def _():
        run(hs0, agg0)

    @pl.when(c == 1)
    def _():
        run(hs1, agg1)


def _tc1_body(x_ref, d0_ref, d1_ref, w_ref, hs0_ref, hs1_ref, dis_ref):
    deg = d0_ref[:, 0:1] + d1_ref[:, 0:1] + 1.0
    dis = lax.rsqrt(deg)
    dis_ref[...] = dis
    x = x_ref[...]
    w = w_ref[...]
    h = (x[:, 0:1] * w[0:1, :] + x[:, 1:2] * w[1:2, :] + x[:, 2:3] * w[2:3, :])
    hs = dis * h
    hs0_ref[...] = hs[:, :H]
    hs1_ref[...] = hs[:, H:]


def _tc1_call(x_pad, deg0, deg1, W1):
    return pl.pallas_call(
        _tc1_body,
        grid=(NTILE,),
        in_specs=[
            pl.BlockSpec((RPT, 3), lambda i: (i, 0)),
            pl.BlockSpec((RPT, DEG_W), lambda i: (i, 0)),
            pl.BlockSpec((RPT, DEG_W), lambda i: (i, 0)),
            pl.BlockSpec((3, D), lambda i: (0, 0)),
        ],
        out_specs=[
            pl.BlockSpec((RPT, H), lambda i: (i, 0)),
            pl.BlockSpec((RPT, H), lambda i: (i, 0)),
            pl.BlockSpec((RPT, 1), lambda i: (i, 0)),
        ],
        out_shape=[
            jax.ShapeDtypeStruct((NROW, H), f32),
            jax.ShapeDtypeStruct((NROW, H), f32),
            jax.ShapeDtypeStruct((NROW, 1), f32),
        ],
    )(x_pad, deg0, deg1, W1)


def _tcmid_body(a0_ref, a1_ref, dis_ref, b_ref, w_ref, hs0_ref, hs1_ref):
    dis = dis_ref[...]
    h0 = jnp.maximum(dis * a0_ref[...] + b_ref[0:1, :H], 0.0)
    h1 = jnp.maximum(dis * a1_ref[...] + b_ref[0:1, H:], 0.0)
    h = jnp.concatenate([h0, h1], axis=1)
    t = jnp.dot(h, w_ref[...], preferred_element_type=f32)
    hs = dis * t
    hs0_ref[...] = hs[:, :H]
    hs1_ref[...] = hs[:, H:]


def _tcmid_call(a0, a1, dis, b_prev, W_next):
    return pl.pallas_call(
        _tcmid_body,
        grid=(NTILE,),
        in_specs=[
            pl.BlockSpec((RPT, H), lambda i: (i, 0)),
            pl.BlockSpec((RPT, H), lambda i: (i, 0)),
            pl.BlockSpec((RPT, 1), lambda i: (i, 0)),
            pl.BlockSpec((1, D), lambda i: (0, 0)),
            pl.BlockSpec((D, D), lambda i: (0, 0)),
        ],
        out_specs=[
            pl.BlockSpec((RPT, H), lambda i: (i, 0)),
            pl.BlockSpec((RPT, H), lambda i: (i, 0)),
        ],
        out_shape=[
            jax.ShapeDtypeStruct((NROW, H), f32),
            jax.ShapeDtypeStruct((NROW, H), f32),
        ],
    )(a0, a1, dis, b_prev, W_next)


def _tcfin_body(a0_ref, a1_ref, dis_ref, b_ref, wfc_ref, bfc_ref, out_ref,
                acc_ref):
    i = pl.program_id(0)
    dis = dis_ref[...]
    h0 = jnp.maximum(dis * a0_ref[...] + b_ref[0:1, :H], 0.0)
    h1 = jnp.maximum(dis * a1_ref[...] + b_ref[0:1, H:], 0.0)
    h = jnp.concatenate([h0, h1], axis=1)
    row = i * RPT + lax.broadcasted_iota(jnp.int32, (RPT, 1), 0)
    h = jnp.where(row < N, h, 0.0)
    part = jnp.sum(h, axis=0, keepdims=True)

    @pl.when(i == 0)
    def _():
        acc_ref[...] = jnp.zeros_like(acc_ref)

    acc_ref[0:1, :] += part

    @pl.when(i == NTILE - 1)
    def _():
        pooled = acc_ref[0:1, :] * (1.0 / N)
        z = jnp.dot(pooled, wfc_ref[...], preferred_element_type=f32)
        out_ref[...] = jnp.tanh(z + bfc_ref[...])


def _tcfin_call(a0, a1, dis, b3, Wfc, bfc):
    return pl.pallas_call(
        _tcfin_body,
        grid=(NTILE,),
        in_specs=[
            pl.BlockSpec((RPT, H), lambda i: (i, 0)),
            pl.BlockSpec((RPT, H), lambda i: (i, 0)),
            pl.BlockSpec((RPT, 1), lambda i: (i, 0)),
            pl.BlockSpec((1, D), lambda i: (0, 0)),
            pl.BlockSpec((D, 24), lambda i: (0, 0)),
            pl.BlockSpec((1, 24), lambda i: (0, 0)),
        ],
        out_specs=pl.BlockSpec((1, 24), lambda i: (0, 0)),
        out_shape=jax.ShapeDtypeStruct((1, 24), f32),
        scratch_shapes=[pltpu.VMEM((1, D), f32)],
    )(a0, a1, dis, b3, Wfc, bfc)


def kernel(x, edge_index, batch, W1, b1, W2, b2, W3, b3, Wfc, bfc):
    src = edge_index[0]
    dst = edge_index[1]
    ar = jnp.arange(PAD, dtype=jnp.int32)
    pad_src = (ar * 97) % N
    pad_dst = N + (ar % NDUM)
    src2d = jnp.concatenate([src, pad_src]).reshape(NCHUNK, K)
    dst2d = jnp.concatenate([dst, pad_dst]).reshape(NCHUNK, K)

    ones_hbm = jnp.ones((K, DEG_W), f32)
    zeros_hbm = jnp.zeros((NROW, DEG_W), f32)
    deg0, deg1 = _deg_kernel(dst2d, ones_hbm, zeros_hbm)

    x_pad = jnp.pad(x, ((0, NROW - N), (0, 0)))
    b1r = b1.reshape(1, D)
    b2r = b2.reshape(1, D)
    b3r = b3.reshape(1, D)
    bfcr = bfc.reshape(1, 24)

    hs0, hs1, dis = _tc1_call(x_pad, deg0, deg1, W1)
    a0, a1 = _agg_kernel(hs0, hs1, src2d, dst2d)
    hs0, hs1 = _tcmid_call(a0, a1, dis, b1r, W2)
    a0, a1 = _agg_kernel(hs0, hs1, src2d, dst2d)
    hs0, hs1 = _tcmid_call(a0, a1, dis, b2r, W3)
    a0, a1 = _agg_kernel(hs0, hs1, src2d, dst2d)
    return _tcfin_call(a0, a1, dis, b3r, Wfc, bfcr)


# speedup vs baseline: 26.1096x; 1.6585x over previous
import functools

import jax
import jax.numpy as jnp
from jax import lax
from jax.experimental import pallas as pl
from jax.experimental.pallas import tpu as pltpu
from jax.experimental.pallas import tpu_sc as plsc

N = 50000
E = 800000
D = 64
H = 32
NDUM = 48
NROW = N + NDUM
NTILE = 16
NCORE = 2
RPT = NROW // NTILE
K = 128
NCHUNK = 6400
EPAD = NCHUNK * K
PAD = EPAD - E
CH = 8
KCH = 4
IG = 40
DEG_W = 8

_mesh = plsc.VectorSubcoreMesh(
    core_axis_name="c", subcore_axis_name="s",
    num_cores=NCORE, num_subcores=NTILE)

_sc_params = pltpu.CompilerParams(use_tc_tiling_on_sc=False)

f32 = jnp.float32


@functools.partial(
    pl.kernel,
    out_type=(jax.ShapeDtypeStruct((NROW, DEG_W), f32),
              jax.ShapeDtypeStruct((NROW, DEG_W), f32)),
    mesh=_mesh,
    scratch_types=[
        pltpu.VMEM((CH, K), jnp.int32),
        pltpu.VMEM((K, DEG_W), f32),
        pltpu.VMEM_SHARED((NROW, DEG_W), f32),
        pltpu.SemaphoreType.DMA,
    ],
    compiler_params=_sc_params,
)
def _deg_kernel(dst2d, ones_hbm, zeros_hbm, deg0, deg1, dstv, onesv, degsh,
                dsem):
    c = lax.axis_index("c")
    s = lax.axis_index("s")
    base = s * RPT
    pltpu.sync_copy(zeros_hbm.at[pl.ds(base, RPT)], degsh.at[pl.ds(base, RPT)])
    pltpu.sync_copy(ones_hbm, onesv)
    plsc.subcore_barrier()

    wid = c * NTILE + s
    chunks_per_w = NCHUNK // (NCORE * NTILE)
    first = wid * chunks_per_w

    def outer(i, carry):
        blk = first + i * CH
        pltpu.sync_copy(dst2d.at[pl.ds(blk, CH)], dstv)
        descs = [pltpu.async_copy(onesv, degsh.at[dstv.at[j]], dsem, add=True)
                 for j in range(CH)]
        for d in descs:
            d.wait()
        return carry

    lax.fori_loop(0, chunks_per_w // CH, outer, 0)
    plsc.subcore_barrier()

    @pl.when(c == 0)
    def _():
        pltpu.sync_copy(degsh.at[pl.ds(base, RPT)], deg0.at[pl.ds(base, RPT)])

    @pl.when(c == 1)
    def _():
        pltpu.sync_copy(degsh.at[pl.ds(base, RPT)], deg1.at[pl.ds(base, RPT)])


@functools.partial(
    pl.kernel,
    out_type=(jax.ShapeDtypeStruct((NROW, H), f32),
              jax.ShapeDtypeStruct((NROW, H), f32)),
    mesh=_mesh,
    scratch_types=[
        pltpu.VMEM((IG, K), jnp.int32),
        pltpu.VMEM((IG, K), jnp.int32),
        pltpu.VMEM((KCH, K, H), f32),
        pltpu.VMEM_SHARED((NROW, H), f32),
        pltpu.SemaphoreType.DMA((KCH,)),
        pltpu.SemaphoreType.DMA,
    ],
    compiler_params=_sc_params,
)
def _agg_kernel(hs0, hs1, src2d, dst2d, agg0, agg1, srcv, dstv, rows, aggsh,
                gsem, ssem):
    c = lax.axis_index("c")
    s = lax.axis_index("s")
    base = s * RPT
    chunks_per_t = NCHUNK // NTILE
    first = s * chunks_per_t

    def run(hs, out):
        pltpu.sync_copy(hs.at[pl.ds(base, RPT)], aggsh.at[pl.ds(base, RPT)])
        plsc.subcore_barrier()

        def outer(i, carry):
            blk = first + i * IG
            pltpu.sync_copy(src2d.at[pl.ds(blk, IG)], srcv)
            pltpu.sync_copy(dst2d.at[pl.ds(blk, IG)], dstv)

            def mid(m, carry2):
                q = m * KCH
                gd = [pltpu.async_copy(hs.at[srcv.at[q + j]], rows.at[j],
                                       gsem.at[j])
                      for j in range(KCH)]
                sd = []
                for j in range(KCH):
                    gd[j].wait()
                    sd.append(pltpu.async_copy(
                        rows.at[j], aggsh.at[dstv.at[q + j]], ssem, add=True))
                for d in sd:
                    d.wait()
                return carry2

            lax.fori_loop(0, IG // KCH, mid, 0)
            return carry

        lax.fori_loop(0, chunks_per_t // IG, outer, 0)
        plsc.subcore_barrier()
        pltpu.sync_copy(aggsh.at[pl.ds(base, RPT)], out.at[pl.ds(base, RPT)])

    @pl.when(c == 0)
    def _():
        run(hs0, agg0)

    @pl.when(c == 1)
    def _():
        run(hs1, agg1)


def _tc1_body(x_ref, d0_ref, d1_ref, w_ref, hs0_ref, hs1_ref, dis_ref):
    deg = d0_ref[:, 0:1] + d1_ref[:, 0:1] + 1.0
    dis = lax.rsqrt(deg)
    dis_ref[...] = dis
    x = x_ref[...]
    w = w_ref[...]
    h = (x[:, 0:1] * w[0:1, :] + x[:, 1:2] * w[1:2, :] + x[:, 2:3] * w[2:3, :])
    hs = dis * h
    hs0_ref[...] = hs[:, :H]
    hs1_ref[...] = hs[:, H:]


def _tc1_call(x_pad, deg0, deg1, W1):
    return pl.pallas_call(
        _tc1_body,
        grid=(NTILE,),
        in_specs=[
            pl.BlockSpec((RPT, 3), lambda i: (i, 0)),
            pl.BlockSpec((RPT, DEG_W), lambda i: (i, 0)),
            pl.BlockSpec((RPT, DEG_W), lambda i: (i, 0)),
            pl.BlockSpec((3, D), lambda i: (0, 0)),
        ],
        out_specs=[
            pl.BlockSpec((RPT, H), lambda i: (i, 0)),
            pl.BlockSpec((RPT, H), lambda i: (i, 0)),
            pl.BlockSpec((RPT, 1), lambda i: (i, 0)),
        ],
        out_shape=[
            jax.ShapeDtypeStruct((NROW, H), f32),
            jax.ShapeDtypeStruct((NROW, H), f32),
            jax.ShapeDtypeStruct((NROW, 1), f32),
        ],
    )(x_pad, deg0, deg1, W1)


def _tcmid_body(a0_ref, a1_ref, dis_ref, b_ref, w_ref, hs0_ref, hs1_ref):
    dis = dis_ref[...]
    h0 = jnp.maximum(dis * a0_ref[...] + b_ref[0:1, :H], 0.0)
    h1 = jnp.maximum(dis * a1_ref[...] + b_ref[0:1, H:], 0.0)
    h = jnp.concatenate([h0, h1], axis=1)
    t = jnp.dot(h, w_ref[...], preferred_element_type=f32)
    hs = dis * t
    hs0_ref[...] = hs[:, :H]
    hs1_ref[...] = hs[:, H:]


def _tcmid_call(a0, a1, dis, b_prev, W_next):
    return pl.pallas_call(
        _tcmid_body,
        grid=(NTILE,),
        in_specs=[
            pl.BlockSpec((RPT, H), lambda i: (i, 0)),
            pl.BlockSpec((RPT, H), lambda i: (i, 0)),
            pl.BlockSpec((RPT, 1), lambda i: (i, 0)),
            pl.BlockSpec((1, D), lambda i: (0, 0)),
            pl.BlockSpec((D, D), lambda i: (0, 0)),
        ],
        out_specs=[
            pl.BlockSpec((RPT, H), lambda i: (i, 0)),
            pl.BlockSpec((RPT, H), lambda i: (i, 0)),
        ],
        out_shape=[
            jax.ShapeDtypeStruct((NROW, H), f32),
            jax.ShapeDtypeStruct((NROW, H), f32),
        ],
    )(a0, a1, dis, b_prev, W_next)


def _tcfin_body(a0_ref, a1_ref, dis_ref, b_ref, wfc_ref, bfc_ref, out_ref,
                acc_ref):
    i = pl.program_id(0)
    dis = dis_ref[...]
    h0 = jnp.maximum(dis * a0_ref[...] + b_ref[0:1, :H], 0.0)
    h1 = jnp.maximum(dis * a1_ref[...] + b_ref[0:1, H:], 0.0)
    h = jnp.concatenate([h0, h1], axis=1)
    row = i * RPT + lax.broadcasted_iota(jnp.int32, (RPT, 1), 0)
    h = jnp.where(row < N, h, 0.0)
    part = jnp.sum(h, axis=0, keepdims=True)

    @pl.when(i == 0)
    def _():
        acc_ref[...] = jnp.zeros_like(acc_ref)

    acc_ref[0:1, :] += part

    @pl.when(i == NTILE - 1)
    def _():
        pooled = acc_ref[0:1, :] * (1.0 / N)
        z = jnp.dot(pooled, wfc_ref[...], preferred_element_type=f32)
        out_ref[...] = jnp.tanh(z + bfc_ref[...])


def _tcfin_call(a0, a1, dis, b3, Wfc, bfc):
    return pl.pallas_call(
        _tcfin_body,
        grid=(NTILE,),
        in_specs=[
            pl.BlockSpec((RPT, H), lambda i: (i, 0)),
            pl.BlockSpec((RPT, H), lambda i: (i, 0)),
            pl.BlockSpec((RPT, 1), lambda i: (i, 0)),
            pl.BlockSpec((1, D), lambda i: (0, 0)),
            pl.BlockSpec((D, 24), lambda i: (0, 0)),
            pl.BlockSpec((1, 24), lambda i: (0, 0)),
        ],
        out_specs=pl.BlockSpec((1, 24), lambda i: (0, 0)),
        out_shape=jax.ShapeDtypeStruct((1, 24), f32),
        scratch_shapes=[pltpu.VMEM((1, D), f32)],
    )(a0, a1, dis, b3, Wfc, bfc)


def kernel(x, edge_index, batch, W1, b1, W2, b2, W3, b3, Wfc, bfc):
    src = edge_index[0]
    dst = edge_index[1]
    ar = jnp.arange(PAD, dtype=jnp.int32)
    pad_src = (ar * 97) % N
    pad_dst = N + (ar % NDUM)
    src2d = jnp.concatenate([src, pad_src]).reshape(NCHUNK, K)
    dst2d = jnp.concatenate([dst, pad_dst]).reshape(NCHUNK, K)

    ones_hbm = jnp.ones((K, DEG_W), f32)
    zeros_hbm = jnp.zeros((NROW, DEG_W), f32)
    deg0, deg1 = _deg_kernel(dst2d, ones_hbm, zeros_hbm)

    x_pad = jnp.pad(x, ((0, NROW - N), (0, 0)))
    b1r = b1.reshape(1, D)
    b2r = b2.reshape(1, D)
    b3r = b3.reshape(1, D)
    bfcr = bfc.reshape(1, 24)

    hs0, hs1, dis = _tc1_call(x_pad, deg0, deg1, W1)
    a0, a1 = _agg_kernel(hs0, hs1, src2d, dst2d)
    hs0, hs1 = _tcmid_call(a0, a1, dis, b1r, W2)
    a0, a1 = _agg_kernel(hs0, hs1, src2d, dst2d)
    hs0, hs1 = _tcmid_call(a0, a1, dis, b2r, W3)
    a0, a1 = _agg_kernel(hs0, hs1, src2d, dst2d)
    return _tcfin_call(a0, a1, dis, b3r, Wfc, bfcr)
